# baseline (speedup 1.0000x reference)
import functools

import jax
import jax.numpy as jnp
import numpy as np
from jax.experimental import pallas as pl
from jax.experimental.pallas import tpu as pltpu

_SPLIT = 1024


def _first_argmin(scores, mind, base):
    K = scores.shape[1]
    iota = jax.lax.broadcasted_iota(jnp.int32, scores.shape, 1)
    return jnp.min(jnp.where(scores <= mind, iota, K), axis=1) + base


def _vq_body(h_ref, cb_ref, tok_ref, q_ref, loss_ref, cn_ref, cbb_ref,
             *, K, split):
    @pl.when(jnp.logical_and(pl.program_id(0) == 0, pl.program_id(1) == 0))
    def _init():
        cbf = cb_ref[...]
        cn_ref[...] = jnp.sum(cbf * cbf, axis=-1)[None, :]
        cbb_ref[...] = (-2.0 * cbf).astype(jnp.bfloat16)

    hb = h_ref[0]
    lat = hb.T
    sq = jnp.sum(lat * lat, axis=-1, keepdims=True)
    cn = cn_ref[...]
    mm = jax.lax.dot_general(lat.astype(jnp.bfloat16), cbb_ref[...],
                             (((1,), (1,)), ((), ())),
                             preferred_element_type=jnp.float32)
    scores = sq + mm + cn
    if split:
        s1 = scores[:, :_SPLIT]
        s2 = scores[:, _SPLIT:]
        m1 = jnp.min(s1, axis=1, keepdims=True)
        i1 = _first_argmin(s1, m1, 0)
        m2 = jnp.min(s2, axis=1, keepdims=True)
        i2 = _first_argmin(s2, m2, _SPLIT)
        m1r = m1.astype(jnp.bfloat16).astype(jnp.float32)
        tok = jnp.where(m2[:, 0] < m1r[:, 0], i2, i1)
    else:
        mind = jnp.min(scores, axis=1, keepdims=True)
        tok = _first_argmin(scores, mind, 0)
    tok_ref[0, 0, :] = tok
    iota = jax.lax.broadcasted_iota(jnp.int32, scores.shape, 1)
    onehot = (iota == tok[:, None]).astype(jnp.float32)
    deqT = jax.lax.dot_general(cb_ref[...], onehot, (((0,), (1,)), ((), ())),
                               preferred_element_type=jnp.float32)
    q_ref[0] = deqT
    diff = deqT - hb
    loss_ref[0, 0, :] = jnp.broadcast_to(jnp.sum(diff * diff), (128,))


def _vq_level(h, cb, tile_t, split):
    B, D, T = h.shape
    K = cb.shape[0]
    TT = min(T, tile_t)
    G = T // TT
    grid = (B, G)
    tok3, q, part = pl.pallas_call(
        functools.partial(_vq_body, K=K, split=split),
        grid=grid,
        in_specs=[
            pl.BlockSpec((1, D, TT), lambda b, t: (b, 0, t)),
            pl.BlockSpec((K, D), lambda b, t: (0, 0)),
        ],
        out_specs=[
            pl.BlockSpec((1, 1, TT), lambda b, t: (b * G + t, 0, 0)),
            pl.BlockSpec((1, D, TT), lambda b, t: (b, 0, t)),
            pl.BlockSpec((1, 1, 128), lambda b, t: (b * G + t, 0, 0)),
        ],
        out_shape=[
            jax.ShapeDtypeStruct((B * G, 1, TT), jnp.int32),
            jax.ShapeDtypeStruct((B, D, T), jnp.float32),
            jax.ShapeDtypeStruct((B * G, 1, 128), jnp.float32),
        ],
        scratch_shapes=[
            pltpu.VMEM((1, K), jnp.float32),
            pltpu.VMEM((K, D), jnp.bfloat16),
        ],
    )(h, cb)
    tokens = tok3.reshape(B, T)
    total = jnp.sum(part[:, 0, 0])
    loss = jnp.sqrt(total) ** 2 / float(np.prod((B * T, D)))
    return tokens, q, loss


def kernel(hidden_states_0, hidden_states_1, hidden_states_2,
           codebook_0, codebook_1, codebook_2):
    t0, q0, l0 = _vq_level(hidden_states_0, codebook_2, 1024, True)
    t1, q1, l1 = _vq_level(hidden_states_1, codebook_1, 1024, False)
    t2, q2, l2 = _vq_level(hidden_states_2, codebook_0, 256, False)
    return (t0, t1, t2, q0, q1, q2, l0, l1, l2)


# speedup vs baseline: 1.0185x; 1.0185x over previous
import functools

import jax
import jax.numpy as jnp
import numpy as np
from jax.experimental import pallas as pl
from jax.experimental.pallas import tpu as pltpu

_SPLIT = 1024


def _first_argmin(scores, mind, base):
    K = scores.shape[1]
    iota = jax.lax.broadcasted_iota(jnp.int32, scores.shape, 1)
    return jnp.min(jnp.where(scores <= mind, iota, K), axis=1) + base


def _vq_body(h_ref, cb_ref, tok_ref, q_ref, loss_ref, cn_ref, cbb_ref,
             *, K, split):
    @pl.when(jnp.logical_and(pl.program_id(0) == 0, pl.program_id(1) == 0))
    def _init():
        cbf = cb_ref[...]
        cn_ref[...] = jnp.sum(cbf * cbf, axis=-1)[None, :]
        cbb_ref[...] = (-2.0 * cbf).astype(jnp.bfloat16)

    hb = h_ref[0]
    lat = hb.T
    sq = jnp.sum(lat * lat, axis=-1, keepdims=True)
    cn = cn_ref[...]
    mm = jax.lax.dot_general(lat.astype(jnp.bfloat16), cbb_ref[...],
                             (((1,), (1,)), ((), ())),
                             preferred_element_type=jnp.float32)
    scores = sq + mm + cn
    if split:
        s1 = scores[:, :_SPLIT]
        s2 = scores[:, _SPLIT:]
        m1 = jnp.min(s1, axis=1, keepdims=True)
        i1 = _first_argmin(s1, m1, 0)
        m2 = jnp.min(s2, axis=1, keepdims=True)
        i2 = _first_argmin(s2, m2, _SPLIT)
        m1r = m1.astype(jnp.bfloat16).astype(jnp.float32)
        take2 = m2[:, 0] < m1r[:, 0]
        tok = jnp.where(take2, i2, i1)
        msel = jnp.where(take2, m2[:, 0], m1[:, 0])
    else:
        mind = jnp.min(scores, axis=1, keepdims=True)
        tok = _first_argmin(scores, mind, 0)
        msel = mind[:, 0]
    tok_ref[0, 0, :] = tok
    iota = jax.lax.broadcasted_iota(jnp.int32, scores.shape, 1)
    onehot = (iota == tok[:, None]).astype(jnp.float32)
    deqT = jax.lax.dot_general(cb_ref[...], onehot, (((0,), (1,)), ((), ())),
                               preferred_element_type=jnp.float32)
    q_ref[0] = deqT
    loss_ref[0, 0, :] = jnp.broadcast_to(jnp.sum(msel), (128,))


def _vq_level(h, cb, tile_t, split):
    B, D, T = h.shape
    K = cb.shape[0]
    TT = min(T, tile_t)
    G = T // TT
    grid = (B, G)
    tok3, q, part = pl.pallas_call(
        functools.partial(_vq_body, K=K, split=split),
        grid=grid,
        in_specs=[
            pl.BlockSpec((1, D, TT), lambda b, t: (b, 0, t)),
            pl.BlockSpec((K, D), lambda b, t: (0, 0)),
        ],
        out_specs=[
            pl.BlockSpec((1, 1, TT), lambda b, t: (b * G + t, 0, 0)),
            pl.BlockSpec((1, D, TT), lambda b, t: (b, 0, t)),
            pl.BlockSpec((1, 1, 128), lambda b, t: (b * G + t, 0, 0)),
        ],
        out_shape=[
            jax.ShapeDtypeStruct((B * G, 1, TT), jnp.int32),
            jax.ShapeDtypeStruct((B, D, T), jnp.float32),
            jax.ShapeDtypeStruct((B * G, 1, 128), jnp.float32),
        ],
        scratch_shapes=[
            pltpu.VMEM((1, K), jnp.float32),
            pltpu.VMEM((K, D), jnp.bfloat16),
        ],
    )(h, cb)
    tokens = tok3.reshape(B, T)
    total = jnp.sum(part[:, 0, 0])
    loss = jnp.sqrt(total) ** 2 / float(np.prod((B * T, D)))
    return tokens, q, loss


def kernel(hidden_states_0, hidden_states_1, hidden_states_2,
           codebook_0, codebook_1, codebook_2):
    t0, q0, l0 = _vq_level(hidden_states_0, codebook_2, 1024, True)
    t1, q1, l1 = _vq_level(hidden_states_1, codebook_1, 1024, False)
    t2, q2, l2 = _vq_level(hidden_states_2, codebook_0, 256, False)
    return (t0, t1, t2, q0, q1, q2, l0, l1, l2)


# speedup vs baseline: 1.0259x; 1.0073x over previous
import functools

import jax
import jax.numpy as jnp
import numpy as np
from jax.experimental import pallas as pl
from jax.experimental.pallas import tpu as pltpu

_SPLIT = 1024


def _first_argmin(scores, mind, base):
    K = scores.shape[1]
    iota = jax.lax.broadcasted_iota(jnp.int32, scores.shape, 1)
    return jnp.min(jnp.where(scores <= mind, iota, K), axis=1) + base


def _vq_body(h_ref, cb_ref, tok_ref, q_ref, loss_ref, cn_ref, cbb_ref,
             *, K, split):
    @pl.when(jnp.logical_and(pl.program_id(0) == 0, pl.program_id(1) == 0))
    def _init():
        cbf = cb_ref[...]
        cn_ref[...] = jnp.sum(cbf * cbf, axis=-1)[None, :]
        cbb_ref[...] = (-2.0 * cbf).astype(jnp.bfloat16)

    hb = h_ref[0]
    lat = hb.T
    sq = jnp.sum(lat * lat, axis=-1, keepdims=True)
    cn = cn_ref[...]
    mm = jax.lax.dot_general(lat.astype(jnp.bfloat16), cbb_ref[...],
                             (((1,), (1,)), ((), ())),
                             preferred_element_type=jnp.float32)
    scores = sq + mm + cn
    if split:
        s1 = scores[:, :_SPLIT]
        s2 = scores[:, _SPLIT:]
        m1 = jnp.min(s1, axis=1, keepdims=True)
        i1 = _first_argmin(s1, m1, 0)
        m2 = jnp.min(s2, axis=1, keepdims=True)
        i2 = _first_argmin(s2, m2, _SPLIT)
        m1r = m1.astype(jnp.bfloat16).astype(jnp.float32)
        take2 = m2[:, 0] < m1r[:, 0]
        tok = jnp.where(take2, i2, i1)
        msel = jnp.where(take2, m2[:, 0], m1[:, 0])
    else:
        mind = jnp.min(scores, axis=1, keepdims=True)
        tok = _first_argmin(scores, mind, 0)
        msel = mind[:, 0]
    tok_ref[0, 0, :] = tok
    iota = jax.lax.broadcasted_iota(jnp.int32, scores.shape, 1)
    onehot = (iota == tok[:, None]).astype(jnp.float32)
    deqT = jax.lax.dot_general(cb_ref[...], onehot, (((0,), (1,)), ((), ())),
                               preferred_element_type=jnp.float32)
    q_ref[0] = deqT
    loss_ref[0, 0, :] = jnp.broadcast_to(jnp.sum(msel), (128,))


def _vq_level(h, cb, tile_t, split):
    B, D, T = h.shape
    K = cb.shape[0]
    TT = min(T, tile_t)
    G = T // TT
    grid = (B, G)
    tok3, q, part = pl.pallas_call(
        functools.partial(_vq_body, K=K, split=split),
        grid=grid,
        in_specs=[
            pl.BlockSpec((1, D, TT), lambda b, t: (b, 0, t)),
            pl.BlockSpec((K, D), lambda b, t: (0, 0)),
        ],
        out_specs=[
            pl.BlockSpec((1, 1, TT), lambda b, t: (b * G + t, 0, 0)),
            pl.BlockSpec((1, D, TT), lambda b, t: (b, 0, t)),
            pl.BlockSpec((1, 1, 128), lambda b, t: (b * G + t, 0, 0)),
        ],
        out_shape=[
            jax.ShapeDtypeStruct((B * G, 1, TT), jnp.int32),
            jax.ShapeDtypeStruct((B, D, T), jnp.float32),
            jax.ShapeDtypeStruct((B * G, 1, 128), jnp.float32),
        ],
        scratch_shapes=[
            pltpu.VMEM((1, K), jnp.float32),
            pltpu.VMEM((K, D), jnp.bfloat16),
        ],
    )(h, cb)
    tokens = tok3.reshape(B, T)
    total = jnp.sum(part[:, 0, 0])
    loss = jnp.sqrt(total) ** 2 / float(np.prod((B * T, D)))
    return tokens, q, loss


def kernel(hidden_states_0, hidden_states_1, hidden_states_2,
           codebook_0, codebook_1, codebook_2):
    t0, q0, l0 = _vq_level(hidden_states_0, codebook_2, 2048, True)
    t1, q1, l1 = _vq_level(hidden_states_1, codebook_1, 1024, False)
    t2, q2, l2 = _vq_level(hidden_states_2, codebook_0, 256, False)
    return (t0, t1, t2, q0, q1, q2, l0, l1, l2)


# speedup vs baseline: 1.0869x; 1.0595x over previous
import jax
import jax.numpy as jnp
import numpy as np
from jax.experimental import pallas as pl
from jax.experimental.pallas import tpu as pltpu

_SPLIT = 1024
_K = 2048
_D = 256
_G0 = 4
_TT0 = 1024


def _first_argmin(scores, mind, base):
    K = scores.shape[1]
    iota = jax.lax.broadcasted_iota(jnp.int32, scores.shape, 1)
    return jnp.min(jnp.where(scores <= mind, iota, K), axis=1) + base


def _level_compute(hb, cb_ref, cn_ref, cbb_ref, tok_ref, q_ref, loss_ref,
                   split):
    lat = hb.T
    sq = jnp.sum(lat * lat, axis=-1, keepdims=True)
    cn = cn_ref[...]
    mm = jax.lax.dot_general(lat.astype(jnp.bfloat16), cbb_ref[...],
                             (((1,), (1,)), ((), ())),
                             preferred_element_type=jnp.float32)
    scores = sq + mm + cn
    if split:
        s1 = scores[:, :_SPLIT]
        s2 = scores[:, _SPLIT:]
        m1 = jnp.min(s1, axis=1, keepdims=True)
        i1 = _first_argmin(s1, m1, 0)
        m2 = jnp.min(s2, axis=1, keepdims=True)
        i2 = _first_argmin(s2, m2, _SPLIT)
        m1r = m1.astype(jnp.bfloat16).astype(jnp.float32)
        take2 = m2[:, 0] < m1r[:, 0]
        tok = jnp.where(take2, i2, i1)
        msel = jnp.where(take2, m2[:, 0], m1[:, 0])
    else:
        mind = jnp.min(scores, axis=1, keepdims=True)
        tok = _first_argmin(scores, mind, 0)
        msel = mind[:, 0]
    tok_ref[0, 0, :] = tok
    iota = jax.lax.broadcasted_iota(jnp.int32, scores.shape, 1)
    onehot = (iota == tok[:, None]).astype(jnp.float32)
    deqT = jax.lax.dot_general(cb_ref[...], onehot, (((0,), (1,)), ((), ())),
                               preferred_element_type=jnp.float32)
    q_ref[0] = deqT
    loss_ref[0, 0, :] = jnp.broadcast_to(jnp.sum(msel), (128,))


def _vq_body(h0_ref, h1_ref, h2_ref, cbl0_ref, cbl1_ref, cbl2_ref,
             tok0_ref, tok1_ref, tok2_ref, q0_ref, q1_ref, q2_ref,
             l0_ref, l1_ref, l2_ref,
             cn0_ref, cbb0_ref, cn1_ref, cbb1_ref, cn2_ref, cbb2_ref):
    b = pl.program_id(0)
    j = pl.program_id(1)

    @pl.when(jnp.logical_and(b == 0, j == 0))
    def _init():
        for cb_ref, cn_ref, cbb_ref in ((cbl0_ref, cn0_ref, cbb0_ref),
                                        (cbl1_ref, cn1_ref, cbb1_ref),
                                        (cbl2_ref, cn2_ref, cbb2_ref)):
            cbf = cb_ref[...]
            cn_ref[...] = jnp.sum(cbf * cbf, axis=-1)[None, :]
            cbb_ref[...] = (-2.0 * cbf).astype(jnp.bfloat16)

    @pl.when(j < _G0)
    def _lvl0():
        _level_compute(h0_ref[0], cbl0_ref, cn0_ref, cbb0_ref,
                       tok0_ref, q0_ref, l0_ref, split=True)

    @pl.when(j == _G0)
    def _lvl1():
        _level_compute(h1_ref[0], cbl1_ref, cn1_ref, cbb1_ref,
                       tok1_ref, q1_ref, l1_ref, split=False)

    @pl.when(j == _G0 + 1)
    def _lvl2():
        _level_compute(h2_ref[0], cbl2_ref, cn2_ref, cbb2_ref,
                       tok2_ref, q2_ref, l2_ref, split=False)


def kernel(hidden_states_0, hidden_states_1, hidden_states_2,
           codebook_0, codebook_1, codebook_2):
    B = hidden_states_0.shape[0]
    T0, T1, T2 = (hidden_states_0.shape[2], hidden_states_1.shape[2],
                  hidden_states_2.shape[2])
    grid = (B, _G0 + 2)

    def _j0(j):
        return jnp.minimum(j, _G0 - 1)

    outs = pl.pallas_call(
        _vq_body,
        grid=grid,
        in_specs=[
            pl.BlockSpec((1, _D, _TT0), lambda b, j: (b, 0, _j0(j))),
            pl.BlockSpec((1, _D, T1), lambda b, j: (b, 0, 0)),
            pl.BlockSpec((1, _D, T2), lambda b, j: (b, 0, 0)),
            pl.BlockSpec((_K, _D), lambda b, j: (0, 0)),
            pl.BlockSpec((_K, _D), lambda b, j: (0, 0)),
            pl.BlockSpec((_K, _D), lambda b, j: (0, 0)),
        ],
        out_specs=[
            pl.BlockSpec((1, 1, _TT0), lambda b, j: (b * _G0 + _j0(j), 0, 0)),
            pl.BlockSpec((1, 1, T1), lambda b, j: (b, 0, 0)),
            pl.BlockSpec((1, 1, T2), lambda b, j: (b, 0, 0)),
            pl.BlockSpec((1, _D, _TT0), lambda b, j: (b, 0, _j0(j))),
            pl.BlockSpec((1, _D, T1), lambda b, j: (b, 0, 0)),
            pl.BlockSpec((1, _D, T2), lambda b, j: (b, 0, 0)),
            pl.BlockSpec((1, 1, 128), lambda b, j: (b * _G0 + _j0(j), 0, 0)),
            pl.BlockSpec((1, 1, 128), lambda b, j: (b, 0, 0)),
            pl.BlockSpec((1, 1, 128), lambda b, j: (b, 0, 0)),
        ],
        out_shape=[
            jax.ShapeDtypeStruct((B * _G0, 1, _TT0), jnp.int32),
            jax.ShapeDtypeStruct((B, 1, T1), jnp.int32),
            jax.ShapeDtypeStruct((B, 1, T2), jnp.int32),
            jax.ShapeDtypeStruct((B, _D, T0), jnp.float32),
            jax.ShapeDtypeStruct((B, _D, T1), jnp.float32),
            jax.ShapeDtypeStruct((B, _D, T2), jnp.float32),
            jax.ShapeDtypeStruct((B * _G0, 1, 128), jnp.float32),
            jax.ShapeDtypeStruct((B, 1, 128), jnp.float32),
            jax.ShapeDtypeStruct((B, 1, 128), jnp.float32),
        ],
        scratch_shapes=[
            pltpu.VMEM((1, _K), jnp.float32), pltpu.VMEM((_K, _D), jnp.bfloat16),
            pltpu.VMEM((1, _K), jnp.float32), pltpu.VMEM((_K, _D), jnp.bfloat16),
            pltpu.VMEM((1, _K), jnp.float32), pltpu.VMEM((_K, _D), jnp.bfloat16),
        ],
    )(hidden_states_0, hidden_states_1, hidden_states_2,
      codebook_2, codebook_1, codebook_0)
    tok0, tok1, tok2, q0, q1, q2, p0, p1, p2 = outs

    def _loss(part, n):
        return jnp.sqrt(jnp.sum(part[..., 0])) ** 2 / float(n)

    return (tok0.reshape(B, T0), tok1.reshape(B, T1), tok2.reshape(B, T2),
            q0, q1, q2,
            _loss(p0, B * T0 * _D), _loss(p1, B * T1 * _D),
            _loss(p2, B * T2 * _D))


# speedup vs baseline: 1.1013x; 1.0133x over previous
import jax
import jax.numpy as jnp
import numpy as np
from jax.experimental import pallas as pl
from jax.experimental.pallas import tpu as pltpu

_SPLIT = 1024
_K = 2048
_D = 256
_G0 = 2
_TT0 = 2048


def _first_argmin(scores, mind, base):
    K = scores.shape[1]
    iota = jax.lax.broadcasted_iota(jnp.int32, scores.shape, 1)
    return jnp.min(jnp.where(scores <= mind, iota, K), axis=1) + base


def _level_compute(hb, cb_ref, cn_ref, cbb_ref, tok_ref, q_ref, loss_ref,
                   split):
    lat = hb.T
    sq = jnp.sum(lat * lat, axis=-1, keepdims=True)
    cn = cn_ref[...]
    mm = jax.lax.dot_general(lat.astype(jnp.bfloat16), cbb_ref[...],
                             (((1,), (1,)), ((), ())),
                             preferred_element_type=jnp.float32)
    scores = sq + mm + cn
    if split:
        s1 = scores[:, :_SPLIT]
        s2 = scores[:, _SPLIT:]
        m1 = jnp.min(s1, axis=1, keepdims=True)
        i1 = _first_argmin(s1, m1, 0)
        m2 = jnp.min(s2, axis=1, keepdims=True)
        i2 = _first_argmin(s2, m2, _SPLIT)
        m1r = m1.astype(jnp.bfloat16).astype(jnp.float32)
        take2 = m2[:, 0] < m1r[:, 0]
        tok = jnp.where(take2, i2, i1)
        msel = jnp.where(take2, m2[:, 0], m1[:, 0])
    else:
        mind = jnp.min(scores, axis=1, keepdims=True)
        tok = _first_argmin(scores, mind, 0)
        msel = mind[:, 0]
    tok_ref[0, 0, :] = tok
    iota = jax.lax.broadcasted_iota(jnp.int32, scores.shape, 1)
    onehot = (iota == tok[:, None]).astype(jnp.float32)
    deqT = jax.lax.dot_general(cb_ref[...], onehot, (((0,), (1,)), ((), ())),
                               preferred_element_type=jnp.float32)
    q_ref[0] = deqT
    loss_ref[0, 0, :] = jnp.broadcast_to(jnp.sum(msel), (128,))


def _vq_body(h0_ref, h1_ref, h2_ref, cbl0_ref, cbl1_ref, cbl2_ref,
             tok0_ref, tok1_ref, tok2_ref, q0_ref, q1_ref, q2_ref,
             l0_ref, l1_ref, l2_ref,
             cn0_ref, cbb0_ref, cn1_ref, cbb1_ref, cn2_ref, cbb2_ref):
    b = pl.program_id(0)
    j = pl.program_id(1)

    @pl.when(jnp.logical_and(b == 0, j == 0))
    def _init():
        for cb_ref, cn_ref, cbb_ref in ((cbl0_ref, cn0_ref, cbb0_ref),
                                        (cbl1_ref, cn1_ref, cbb1_ref),
                                        (cbl2_ref, cn2_ref, cbb2_ref)):
            cbf = cb_ref[...]
            cn_ref[...] = jnp.sum(cbf * cbf, axis=-1)[None, :]
            cbb_ref[...] = (-2.0 * cbf).astype(jnp.bfloat16)

    @pl.when(j < _G0)
    def _lvl0():
        _level_compute(h0_ref[0], cbl0_ref, cn0_ref, cbb0_ref,
                       tok0_ref, q0_ref, l0_ref, split=True)

    @pl.when(j == _G0)
    def _lvl1():
        _level_compute(h1_ref[0], cbl1_ref, cn1_ref, cbb1_ref,
                       tok1_ref, q1_ref, l1_ref, split=False)

    @pl.when(j == _G0 + 1)
    def _lvl2():
        _level_compute(h2_ref[0], cbl2_ref, cn2_ref, cbb2_ref,
                       tok2_ref, q2_ref, l2_ref, split=False)


def kernel(hidden_states_0, hidden_states_1, hidden_states_2,
           codebook_0, codebook_1, codebook_2):
    B = hidden_states_0.shape[0]
    T0, T1, T2 = (hidden_states_0.shape[2], hidden_states_1.shape[2],
                  hidden_states_2.shape[2])
    grid = (B, _G0 + 2)

    def _j0(j):
        return jnp.minimum(j, _G0 - 1)

    outs = pl.pallas_call(
        _vq_body,
        grid=grid,
        in_specs=[
            pl.BlockSpec((1, _D, _TT0), lambda b, j: (b, 0, _j0(j))),
            pl.BlockSpec((1, _D, T1), lambda b, j: (b, 0, 0)),
            pl.BlockSpec((1, _D, T2), lambda b, j: (b, 0, 0)),
            pl.BlockSpec((_K, _D), lambda b, j: (0, 0)),
            pl.BlockSpec((_K, _D), lambda b, j: (0, 0)),
            pl.BlockSpec((_K, _D), lambda b, j: (0, 0)),
        ],
        out_specs=[
            pl.BlockSpec((1, 1, _TT0), lambda b, j: (b * _G0 + _j0(j), 0, 0)),
            pl.BlockSpec((1, 1, T1), lambda b, j: (b, 0, 0)),
            pl.BlockSpec((1, 1, T2), lambda b, j: (b, 0, 0)),
            pl.BlockSpec((1, _D, _TT0), lambda b, j: (b, 0, _j0(j))),
            pl.BlockSpec((1, _D, T1), lambda b, j: (b, 0, 0)),
            pl.BlockSpec((1, _D, T2), lambda b, j: (b, 0, 0)),
            pl.BlockSpec((1, 1, 128), lambda b, j: (b * _G0 + _j0(j), 0, 0)),
            pl.BlockSpec((1, 1, 128), lambda b, j: (b, 0, 0)),
            pl.BlockSpec((1, 1, 128), lambda b, j: (b, 0, 0)),
        ],
        out_shape=[
            jax.ShapeDtypeStruct((B * _G0, 1, _TT0), jnp.int32),
            jax.ShapeDtypeStruct((B, 1, T1), jnp.int32),
            jax.ShapeDtypeStruct((B, 1, T2), jnp.int32),
            jax.ShapeDtypeStruct((B, _D, T0), jnp.float32),
            jax.ShapeDtypeStruct((B, _D, T1), jnp.float32),
            jax.ShapeDtypeStruct((B, _D, T2), jnp.float32),
            jax.ShapeDtypeStruct((B * _G0, 1, 128), jnp.float32),
            jax.ShapeDtypeStruct((B, 1, 128), jnp.float32),
            jax.ShapeDtypeStruct((B, 1, 128), jnp.float32),
        ],
        scratch_shapes=[
            pltpu.VMEM((1, _K), jnp.float32), pltpu.VMEM((_K, _D), jnp.bfloat16),
            pltpu.VMEM((1, _K), jnp.float32), pltpu.VMEM((_K, _D), jnp.bfloat16),
            pltpu.VMEM((1, _K), jnp.float32), pltpu.VMEM((_K, _D), jnp.bfloat16),
        ],
    )(hidden_states_0, hidden_states_1, hidden_states_2,
      codebook_2, codebook_1, codebook_0)
    tok0, tok1, tok2, q0, q1, q2, p0, p1, p2 = outs

    def _loss(part, n):
        return jnp.sqrt(jnp.sum(part[..., 0])) ** 2 / float(n)

    return (tok0.reshape(B, T0), tok1.reshape(B, T1), tok2.reshape(B, T2),
            q0, q1, q2,
            _loss(p0, B * T0 * _D), _loss(p1, B * T1 * _D),
            _loss(p2, B * T2 * _D))
